# Initial kernel scaffold; baseline (speedup 1.0000x reference)
import jax
import jax.numpy as jnp
from jax.experimental import pallas as pl


def kernel(emb_region, emb_queue, emb_champ, W1, b1, W2, b2, region, queue_type, champion_ids):
    raise NotImplementedError("write your pallas kernel here")


# speedup vs baseline: 1.7940x; 1.7940x over previous
import functools

import jax
import jax.numpy as jnp
from jax import lax
from jax.experimental import pallas as pl
from jax.experimental.pallas import tpu as pltpu
from jax.experimental.pallas import tpu_sc as plsc

B = 16384
D = 32
NSLOT = 10
NC, NS = 2, 16
NW = NC * NS
BPW = B // NW
CHUNK = 128
NCHUNK = BPW // CHUNK

IDXW = 128


def _gather_body(emb_region, emb_queue, emb_champ, region2d, queue2d, champ2d,
                 xr, xq, xc, idx_r, idx_q, idx_c, rows_r, rows_q, rows_c, sem):
    w = lax.axis_index("s") * NC + lax.axis_index("c")
    pltpu.sync_copy(region2d.at[pl.ds(w * NCHUNK, NCHUNK)], idx_r)
    pltpu.sync_copy(queue2d.at[pl.ds(w * NCHUNK, NCHUNK)], idx_q)
    pltpu.sync_copy(champ2d.at[pl.ds(w * NCHUNK * NSLOT, NCHUNK * NSLOT)], idx_c)

    def chunk_step(c, carry):
        base = w * BPW + c * CHUNK
        copies = [
            pltpu.async_copy(emb_region.at[idx_r.at[c]], rows_r, sem),
            pltpu.async_copy(emb_queue.at[idx_q.at[c]], rows_q, sem),
        ]
        for j in range(NSLOT):
            copies.append(
                pltpu.async_copy(emb_champ.at[idx_c.at[c * NSLOT + j]],
                                 rows_c.at[pl.ds(j * CHUNK, CHUNK)], sem))
        for cp in copies:
            cp.wait()
        pltpu.sync_copy(rows_r, xr.at[pl.ds(base, CHUNK)])
        pltpu.sync_copy(rows_q, xq.at[pl.ds(base, CHUNK)])
        pltpu.sync_copy(rows_c, xc.at[pl.ds(base * NSLOT, CHUNK * NSLOT)])
        return carry

    lax.fori_loop(0, NCHUNK, chunk_step, 0)


def _sc_gather(emb_region, emb_queue, emb_champ, region, queue_type, champ_flat):
    region2d = region.reshape(B // IDXW, IDXW)
    queue2d = queue_type.reshape(B // IDXW, IDXW)
    champ2d = champ_flat.reshape(B * NSLOT // IDXW, IDXW)
    mesh = plsc.VectorSubcoreMesh(core_axis_name="c", subcore_axis_name="s",
                                  num_cores=NC, num_subcores=NS)
    f = pl.kernel(
        _gather_body,
        out_type=(
            jax.ShapeDtypeStruct((B, D), jnp.float32),
            jax.ShapeDtypeStruct((B, D), jnp.float32),
            jax.ShapeDtypeStruct((B * NSLOT, D), jnp.float32),
        ),
        mesh=mesh,
        scratch_types=[
            pltpu.VMEM((NCHUNK, IDXW), jnp.int32),
            pltpu.VMEM((NCHUNK, IDXW), jnp.int32),
            pltpu.VMEM((NCHUNK * NSLOT, IDXW), jnp.int32),
            pltpu.VMEM((CHUNK, D), jnp.float32),
            pltpu.VMEM((CHUNK, D), jnp.float32),
            pltpu.VMEM((CHUNK * NSLOT, D), jnp.float32),
            pltpu.SemaphoreType.DMA,
        ],
        compiler_params=pltpu.CompilerParams(use_tc_tiling_on_sc=False),
    )
    return f(emb_region, emb_queue, emb_champ, region2d, queue2d, champ2d)


MLP_BLK = 2048


def _mlp_body(xr, xq, xc, w1r, w1q, w1c, b1, w2, b2, out):
    h = (jnp.dot(xr[...], w1r[...], preferred_element_type=jnp.float32)
         + jnp.dot(xq[...], w1q[...], preferred_element_type=jnp.float32)
         + jnp.dot(xc[...], w1c[...], preferred_element_type=jnp.float32)
         + b1[...])
    h = jnp.maximum(h, 0.0)
    o = jnp.dot(h, w2[...], preferred_element_type=jnp.float32) + b2[...]
    out[...] = 1.0 / (1.0 + jnp.exp(-o))


def _mlp(xr, xq, xc, W1, b1, W2, b2):
    w1r, w1q, w1c = W1[:D], W1[D:2 * D], W1[2 * D:]
    grid = (B // MLP_BLK,)
    return pl.pallas_call(
        _mlp_body,
        grid=grid,
        in_specs=[
            pl.BlockSpec((MLP_BLK, D), lambda i: (i, 0)),
            pl.BlockSpec((MLP_BLK, D), lambda i: (i, 0)),
            pl.BlockSpec((MLP_BLK, NSLOT * D), lambda i: (i, 0)),
            pl.BlockSpec((D, 128), lambda i: (0, 0)),
            pl.BlockSpec((D, 128), lambda i: (0, 0)),
            pl.BlockSpec((NSLOT * D, 128), lambda i: (0, 0)),
            pl.BlockSpec((1, 128), lambda i: (0, 0)),
            pl.BlockSpec((128, 1), lambda i: (0, 0)),
            pl.BlockSpec((1, 1), lambda i: (0, 0)),
        ],
        out_specs=pl.BlockSpec((MLP_BLK, 1), lambda i: (i, 0)),
        out_shape=jax.ShapeDtypeStruct((B, 1), jnp.float32),
    )(xr, xq, xc, w1r, w1q, w1c, b1.reshape(1, 128), W2, b2.reshape(1, 1))


def kernel(emb_region, emb_queue, emb_champ, W1, b1, W2, b2, region, queue_type, champion_ids):
    region = region.astype(jnp.int32)
    queue_type = queue_type.astype(jnp.int32)
    champ_flat = champion_ids.astype(jnp.int32).reshape(B * NSLOT)
    xr, xq, xc = _sc_gather(emb_region, emb_queue, emb_champ,
                            region, queue_type, champ_flat)
    xc = xc.reshape(B, NSLOT * D)
    out = _mlp(xr, xq, xc, W1, b1, W2, b2)
    return jnp.squeeze(out, axis=1)
